# Initial kernel scaffold; baseline (speedup 1.0000x reference)
import jax
import jax.numpy as jnp
from jax.experimental import pallas as pl


def kernel(x, edge_index, edge_weight, W0, b0, Wconv, Wm, bm, W1, b1):
    raise NotImplementedError("write your pallas kernel here")


# speedup vs baseline: 3.0666x; 3.0666x over previous
import functools

import jax
import jax.numpy as jnp
from jax import lax
from jax.experimental import pallas as pl
from jax.experimental.pallas import tpu as pltpu
from jax.experimental.pallas import tpu_sc as plsc

_N = 10000
_E = 320000
_DIN = 128
_H = 64
_L = 8
_CLS = 40
_ENT = 0.5

_NC = 2
_NS = 16
_NW = _NC * _NS
_CH = 128
_K = 80
_EPT = _K * _CH
_EPAD = _NW * _EPT
_NP = 10240
_RB = _NP // _NS
_RSUB = _RB // _CH


def _input_kernel(x_pad, W0, b0r):
    blk = 512
    grid = _NP // blk

    def body(x_ref, w_ref, b_ref, out_ref):
        z = jnp.dot(x_ref[...], w_ref[...], preferred_element_type=jnp.float32)
        out_ref[0] = z + b_ref[...]
        out_ref[1] = jnp.zeros_like(z)

    return pl.pallas_call(
        body,
        grid=(grid,),
        in_specs=[
            pl.BlockSpec((blk, _DIN), lambda i: (i, 0)),
            pl.BlockSpec((_DIN, _H), lambda i: (0, 0)),
            pl.BlockSpec((1, _H), lambda i: (0, 0)),
        ],
        out_specs=pl.BlockSpec((2, blk, _H), lambda i: (0, i, 0)),
        out_shape=jax.ShapeDtypeStruct((2, _NP, _H), jnp.float32),
    )(x_pad, W0, b0r)


def _make_sc_layer():
    mesh = plsc.VectorSubcoreMesh(core_axis_name="c", subcore_axis_name="s")

    def body(prev_ref, src_ref, dst_ref, w_ref, part_ref, cur_ref,
             src_v, dst_v, w_v, gbuf, pA, pB, sacc, semA, semB):
        cid = lax.axis_index("c")
        sid = lax.axis_index("s")
        wid = cid * _NS + sid

        pltpu.sync_copy(src_ref.at[cid, wid], src_v)
        pltpu.sync_copy(dst_ref.at[wid], dst_v)
        pltpu.sync_copy(w_ref.at[wid], w_v)

        for k in range(_RSUB):
            row0 = sid * _RB + k * _CH
            pltpu.sync_copy(prev_ref.at[0, pl.ds(row0, _CH)], pA)
            pltpu.sync_copy(prev_ref.at[1, pl.ds(row0, _CH)], pB)

            def cmb(r, _):
                for c4 in range(4):
                    sl = pl.ds(c4 * 16, 16)
                    pA[r, sl] = jnp.maximum(pA[r, sl] + pB[r, sl], 0.0)
                return 0

            lax.fori_loop(0, _CH, cmb, 0, unroll=4)
            pltpu.sync_copy(pA, cur_ref.at[pl.ds(cid * _NP + row0, _CH)])

        def zb(r, _):
            for c4 in range(4):
                pB[r, pl.ds(c4 * 16, 16)] = jnp.zeros((16,), jnp.float32)
            return 0

        lax.fori_loop(0, _CH, zb, 0, unroll=4)
        for k in range(_RSUB):
            row0 = sid * _RB + k * _CH
            pltpu.sync_copy(pB, sacc.at[pl.ds(row0, _CH)])

        plsc.subcore_barrier()

        def pair(i, _):
            jA = 2 * i
            jB = jA + 1
            dA = pltpu.async_copy(cur_ref.at[src_v.at[jA]], gbuf.at[0], semA)
            dB = pltpu.async_copy(cur_ref.at[src_v.at[jB]], gbuf.at[1], semB)
            dA.wait()

            def scaleA(g, _):
                wvec = w_v[jA, pl.ds(g * 16, 16)]
                for r16 in range(16):
                    w = wvec[r16]
                    for c4 in range(4):
                        sl = pl.ds(c4 * 16, 16)
                        gbuf[0, g * 16 + r16, sl] = gbuf[0, g * 16 + r16, sl] * w
                return 0

            lax.fori_loop(0, _CH // 16, scaleA, 0)
            pltpu.sync_copy(gbuf.at[0], sacc.at[dst_v.at[jA]], add=True)
            dB.wait()

            def scaleB(g, _):
                wvec = w_v[jB, pl.ds(g * 16, 16)]
                for r16 in range(16):
                    w = wvec[r16]
                    for c4 in range(4):
                        sl = pl.ds(c4 * 16, 16)
                        gbuf[1, g * 16 + r16, sl] = gbuf[1, g * 16 + r16, sl] * w
                return 0

            lax.fori_loop(0, _CH // 16, scaleB, 0)
            pltpu.sync_copy(gbuf.at[1], sacc.at[dst_v.at[jB]], add=True)
            return 0

        lax.fori_loop(0, _K // 2, pair, 0)

        plsc.subcore_barrier()

        pltpu.sync_copy(sacc.at[pl.ds(sid * _RB, _RB)],
                        part_ref.at[cid, pl.ds(sid * _RB, _RB)])

    return pl.kernel(
        body,
        out_type=(jax.ShapeDtypeStruct((_NC, _NP, _H), jnp.float32),
                  jax.ShapeDtypeStruct((_NC * _NP, _H), jnp.float32)),
        mesh=mesh,
        compiler_params=pltpu.CompilerParams(use_tc_tiling_on_sc=False),
        scratch_types=[
            pltpu.VMEM((_K, _CH), jnp.int32),
            pltpu.VMEM((_K, _CH), jnp.int32),
            pltpu.VMEM((_K, _CH), jnp.float32),
            pltpu.VMEM((2, _CH, _H), jnp.float32),
            pltpu.VMEM((_CH, _H), jnp.float32),
            pltpu.VMEM((_CH, _H), jnp.float32),
            pltpu.VMEM_SHARED((_NP, _H), jnp.float32),
            pltpu.SemaphoreType.DMA,
            pltpu.SemaphoreType.DMA,
        ],
    )


_SC_LAYER = _make_sc_layer()


def _final_kernel(xs, part8, eg, WmT, bm2, W1, b1r):
    blk = 1000
    grid = _N // blk

    def body(*refs):
        x_refs = refs[:_L]
        p8_ref, eg_ref, wm_ref, bm_ref, w1_ref, b1_ref, out_ref, ent_ref = refs[_L:]
        i = pl.program_id(0)

        xs8 = jnp.maximum(p8_ref[0] + p8_ref[1], 0.0)
        X = [r[...] for r in x_refs] + [xs8]
        wm = wm_ref[...]
        bm = bm_ref[0, 0]

        s = [1.0 / (1.0 + jnp.exp(-(jnp.sum(Xc * wm, axis=1) + bm))) for Xc in X]

        t = [None] * (_L + 1)
        t[0] = s[0]
        P = 1.0 - s[0]
        for c in range(1, _L):
            t[c] = s[c] * P
            P = P * (1.0 - s[c])
        t[_L] = P

        ent_part = sum(jnp.sum(tc * jnp.log(tc + 1e-20)) for tc in t)

        @pl.when(i == 0)
        def _():
            ent_ref[0, 0] = 0.0

        ent_ref[0, 0] += ent_part

        egm = eg_ref[...]
        u = [(t[c] + 1e-20) * egm[:, c] for c in range(_L + 1)]
        Z = u[0]
        for c in range(1, _L + 1):
            Z = Z + u[c]
        invZ = 1.0 / Z
        xagg = (u[0] * invZ)[:, None] * X[0]
        for c in range(1, _L + 1):
            xagg = xagg + (u[c] * invZ)[:, None] * X[c]

        out = jnp.dot(xagg, w1_ref[...], preferred_element_type=jnp.float32)
        out = out + b1_ref[...]
        m = jnp.max(out, axis=1, keepdims=True)
        lse = jnp.log(jnp.sum(jnp.exp(out - m), axis=1, keepdims=True)) + m
        out_ref[...] = out - lse

    in_specs = (
        [pl.BlockSpec((blk, _H), lambda i: (i, 0)) for _ in range(_L)]
        + [
            pl.BlockSpec((2, blk, _H), lambda i: (0, i, 0)),
            pl.BlockSpec((blk, _L + 1), lambda i: (i, 0)),
            pl.BlockSpec((1, _H), lambda i: (0, 0)),
            pl.BlockSpec((1, 1), lambda i: (0, 0), memory_space=pltpu.SMEM),
            pl.BlockSpec((_H, _CLS), lambda i: (0, 0)),
            pl.BlockSpec((1, _CLS), lambda i: (0, 0)),
        ]
    )
    return pl.pallas_call(
        body,
        grid=(grid,),
        in_specs=in_specs,
        out_specs=(
            pl.BlockSpec((blk, _CLS), lambda i: (i, 0)),
            pl.BlockSpec((1, 1), lambda i: (0, 0), memory_space=pltpu.SMEM),
        ),
        out_shape=(
            jax.ShapeDtypeStruct((_N, _CLS), jnp.float32),
            jax.ShapeDtypeStruct((1, 1), jnp.float32),
        ),
    )(*xs, part8, eg, WmT, bm2, W1, b1r)


def kernel(x, edge_index, edge_weight, W0, b0, Wconv, Wm, bm, W1, b1):
    del Wconv

    src = edge_index[0]
    dst = edge_index[1]
    pad = _EPAD - _E
    srcs = jnp.pad(src, (0, pad)).reshape(_NW, _K, _CH)
    srcs = jnp.stack([srcs, srcs + _NP])
    dsts = jnp.pad(dst, (0, pad)).reshape(_NW, _K, _CH)
    ws = jnp.pad(edge_weight, (0, pad)).reshape(_NW, _K, _CH)

    x_pad = jnp.pad(x, ((0, _NP - _N), (0, 0)))
    part = _input_kernel(x_pad, W0, b0.reshape(1, _H))

    xs = []
    for _ in range(_L):
        part, cur = _SC_LAYER(part, srcs, dsts, ws)
        xs.append(cur)

    g = jax.random.gumbel(jax.random.key(42), (_N, _L + 1), jnp.float32)
    eg = jnp.exp(g)

    logp, ent_sum = _final_kernel(
        xs, part, eg, Wm.reshape(1, _H), bm.reshape(1, 1), W1,
        b1.reshape(1, _CLS),
    )
    ent_loss = _ENT * -(ent_sum[0, 0] / _N)
    return logp, ent_loss


# speedup vs baseline: 4.2342x; 1.3807x over previous
import functools

import jax
import jax.numpy as jnp
from jax import lax
from jax.experimental import pallas as pl
from jax.experimental.pallas import tpu as pltpu
from jax.experimental.pallas import tpu_sc as plsc

_N = 10000
_E = 320000
_DIN = 128
_H = 64
_L = 8
_CLS = 40
_ENT = 0.5

_NC = 2
_NS = 16
_NW = _NC * _NS
_CH = 128
_K = 80
_EPT = _K * _CH
_EPAD = _NW * _EPT
_NP = 10240
_RB = _NP // _NS
_RSUB = _RB // _CH


def _input_kernel(x_pad, W0, b0r):
    blk = 512
    grid = _NP // blk

    def body(x_ref, w_ref, b_ref, out_ref):
        z = jnp.dot(x_ref[...], w_ref[...], preferred_element_type=jnp.float32)
        out_ref[0] = z + b_ref[...]
        out_ref[1] = jnp.zeros_like(z)

    return pl.pallas_call(
        body,
        grid=(grid,),
        in_specs=[
            pl.BlockSpec((blk, _DIN), lambda i: (i, 0)),
            pl.BlockSpec((_DIN, _H), lambda i: (0, 0)),
            pl.BlockSpec((1, _H), lambda i: (0, 0)),
        ],
        out_specs=pl.BlockSpec((2, blk, _H), lambda i: (0, i, 0)),
        out_shape=jax.ShapeDtypeStruct((2, _NP, _H), jnp.float32),
    )(x_pad, W0, b0r)


def _make_sc_layer():
    mesh = plsc.VectorSubcoreMesh(core_axis_name="c", subcore_axis_name="s")

    def body(prev_ref, src_ref, dst_ref, w_ref, part_ref, cur_ref,
             src_v, dst_v, gbuf, wbuf, pA, pB, sacc,
             g0, g1, g2, g3, s0, s1, s2, s3):
        gsems = (g0, g1, g2, g3)
        ssems = (s0, s1, s2, s3)
        cid = lax.axis_index("c")
        sid = lax.axis_index("s")
        wid = cid * _NS + sid

        pltpu.sync_copy(src_ref.at[cid, wid], src_v)
        pltpu.sync_copy(dst_ref.at[wid], dst_v)

        for k in range(_RSUB):
            row0 = sid * _RB + k * _CH
            pltpu.sync_copy(prev_ref.at[0, pl.ds(row0, _CH)], pA)
            pltpu.sync_copy(prev_ref.at[1, pl.ds(row0, _CH)], pB)

            def cmb(r, _):
                for c4 in range(4):
                    sl = pl.ds(c4 * 16, 16)
                    pA[r, sl] = jnp.maximum(pA[r, sl] + pB[r, sl], 0.0)
                return 0

            lax.fori_loop(0, _CH, cmb, 0, unroll=4)
            pltpu.sync_copy(pA, cur_ref.at[pl.ds(cid * _NP + row0, _CH)])

        def zb(r, _):
            for c4 in range(4):
                pB[r, pl.ds(c4 * 16, 16)] = jnp.zeros((16,), jnp.float32)
            return 0

        lax.fori_loop(0, _CH, zb, 0, unroll=4)
        for k in range(_RSUB):
            row0 = sid * _RB + k * _CH
            pltpu.sync_copy(pB, sacc.at[pl.ds(row0, _CH)])

        plsc.subcore_barrier()

        for s in range(4):
            pltpu.async_copy(cur_ref.at[src_v.at[s]], gbuf.at[s], gsems[s])
            pltpu.async_copy(w_ref.at[wid, s], wbuf.at[s], gsems[s])

        def quad(i, _):
            for s in range(4):
                j = 4 * i + s
                pltpu.make_async_copy(
                    cur_ref.at[pl.ds(0, _CH)], gbuf.at[s], gsems[s]).wait()
                pltpu.make_async_copy(
                    w_ref.at[0, 0], wbuf.at[s], gsems[s]).wait()

                def srow(r, _):
                    wrow = wbuf[s, r]
                    for c4 in range(4):
                        sl = pl.ds(c4 * 16, 16)
                        gbuf[s, r, sl] = gbuf[s, r, sl] * wrow
                    return 0

                lax.fori_loop(0, _CH, srow, 0, unroll=4)
                pltpu.async_copy(gbuf.at[s], sacc.at[dst_v.at[j]],
                                 ssems[s], add=True)
            for s in range(4):
                j2 = 4 * i + 4 + s
                pltpu.make_async_copy(
                    cur_ref.at[pl.ds(0, _CH)], gbuf.at[s], ssems[s]).wait()

                @pl.when(j2 < _K)
                def _():
                    pltpu.async_copy(cur_ref.at[src_v.at[j2]], gbuf.at[s],
                                     gsems[s])
                    pltpu.async_copy(w_ref.at[wid, j2], wbuf.at[s], gsems[s])
            return 0

        lax.fori_loop(0, _K // 4, quad, 0)

        plsc.subcore_barrier()

        pltpu.sync_copy(sacc.at[pl.ds(sid * _RB, _RB)],
                        part_ref.at[cid, pl.ds(sid * _RB, _RB)])

    return pl.kernel(
        body,
        out_type=(jax.ShapeDtypeStruct((_NC, _NP, _H), jnp.float32),
                  jax.ShapeDtypeStruct((_NC * _NP, _H), jnp.float32)),
        mesh=mesh,
        compiler_params=pltpu.CompilerParams(use_tc_tiling_on_sc=False),
        scratch_types=[
            pltpu.VMEM((_K, _CH), jnp.int32),
            pltpu.VMEM((_K, _CH), jnp.int32),
            pltpu.VMEM((4, _CH, _H), jnp.float32),
            pltpu.VMEM((4, _CH, 16), jnp.float32),
            pltpu.VMEM((_CH, _H), jnp.float32),
            pltpu.VMEM((_CH, _H), jnp.float32),
            pltpu.VMEM_SHARED((_NP, _H), jnp.float32),
            pltpu.SemaphoreType.DMA,
            pltpu.SemaphoreType.DMA,
            pltpu.SemaphoreType.DMA,
            pltpu.SemaphoreType.DMA,
            pltpu.SemaphoreType.DMA,
            pltpu.SemaphoreType.DMA,
            pltpu.SemaphoreType.DMA,
            pltpu.SemaphoreType.DMA,
        ],
    )


_SC_LAYER = _make_sc_layer()


def _final_kernel(xs, part8, eg, WmT, bm2, W1, b1r):
    blk = 1000
    grid = _N // blk

    def body(*refs):
        x_refs = refs[:_L]
        p8_ref, eg_ref, wm_ref, bm_ref, w1_ref, b1_ref, out_ref, ent_ref = refs[_L:]
        i = pl.program_id(0)

        xs8 = jnp.maximum(p8_ref[0] + p8_ref[1], 0.0)
        X = [r[...] for r in x_refs] + [xs8]
        wm = wm_ref[...]
        bm = bm_ref[0, 0]

        s = [1.0 / (1.0 + jnp.exp(-(jnp.sum(Xc * wm, axis=1) + bm))) for Xc in X]

        t = [None] * (_L + 1)
        t[0] = s[0]
        P = 1.0 - s[0]
        for c in range(1, _L):
            t[c] = s[c] * P
            P = P * (1.0 - s[c])
        t[_L] = P

        ent_part = sum(jnp.sum(tc * jnp.log(tc + 1e-20)) for tc in t)

        @pl.when(i == 0)
        def _():
            ent_ref[0, 0] = 0.0

        ent_ref[0, 0] += ent_part

        egm = eg_ref[...]
        u = [(t[c] + 1e-20) * egm[:, c] for c in range(_L + 1)]
        Z = u[0]
        for c in range(1, _L + 1):
            Z = Z + u[c]
        invZ = 1.0 / Z
        xagg = (u[0] * invZ)[:, None] * X[0]
        for c in range(1, _L + 1):
            xagg = xagg + (u[c] * invZ)[:, None] * X[c]

        out = jnp.dot(xagg, w1_ref[...], preferred_element_type=jnp.float32)
        out = out + b1_ref[...]
        m = jnp.max(out, axis=1, keepdims=True)
        lse = jnp.log(jnp.sum(jnp.exp(out - m), axis=1, keepdims=True)) + m
        out_ref[...] = out - lse

    in_specs = (
        [pl.BlockSpec((blk, _H), lambda i: (i, 0)) for _ in range(_L)]
        + [
            pl.BlockSpec((2, blk, _H), lambda i: (0, i, 0)),
            pl.BlockSpec((blk, _L + 1), lambda i: (i, 0)),
            pl.BlockSpec((1, _H), lambda i: (0, 0)),
            pl.BlockSpec((1, 1), lambda i: (0, 0), memory_space=pltpu.SMEM),
            pl.BlockSpec((_H, _CLS), lambda i: (0, 0)),
            pl.BlockSpec((1, _CLS), lambda i: (0, 0)),
        ]
    )
    return pl.pallas_call(
        body,
        grid=(grid,),
        in_specs=in_specs,
        out_specs=(
            pl.BlockSpec((blk, _CLS), lambda i: (i, 0)),
            pl.BlockSpec((1, 1), lambda i: (0, 0), memory_space=pltpu.SMEM),
        ),
        out_shape=(
            jax.ShapeDtypeStruct((_N, _CLS), jnp.float32),
            jax.ShapeDtypeStruct((1, 1), jnp.float32),
        ),
    )(*xs, part8, eg, WmT, bm2, W1, b1r)


def kernel(x, edge_index, edge_weight, W0, b0, Wconv, Wm, bm, W1, b1):
    del Wconv

    src = edge_index[0]
    dst = edge_index[1]
    pad = _EPAD - _E
    srcs = jnp.pad(src, (0, pad)).reshape(_NW, _K, _CH)
    srcs = jnp.stack([srcs, srcs + _NP])
    dsts = jnp.pad(dst, (0, pad)).reshape(_NW, _K, _CH)
    w_pad = jnp.pad(edge_weight, (0, pad))
    ws = jnp.broadcast_to(w_pad[:, None], (_EPAD, 16)).reshape(
        _NW, _K, _CH, 16)

    x_pad = jnp.pad(x, ((0, _NP - _N), (0, 0)))
    part = _input_kernel(x_pad, W0, b0.reshape(1, _H))

    xs = []
    for _ in range(_L):
        part, cur = _SC_LAYER(part, srcs, dsts, ws)
        xs.append(cur)

    g = jax.random.gumbel(jax.random.key(42), (_N, _L + 1), jnp.float32)
    eg = jnp.exp(g)

    logp, ent_sum = _final_kernel(
        xs, part, eg, Wm.reshape(1, _H), bm.reshape(1, 1), W1,
        b1.reshape(1, _CLS),
    )
    ent_loss = _ENT * -(ent_sum[0, 0] / _N)
    return logp, ent_loss
